# baseline (speedup 1.0000x reference)
import functools

import jax
import jax.numpy as jnp
from jax import lax
from jax.experimental import pallas as pl
from jax.experimental.pallas import tpu as pltpu
from jax.experimental.pallas import tpu_sc as plsc

_L = 100
_D = 128
_LPAD = 128
_B = 1024
_CB = 256
_NQ = _B // _CB
_NTASK = _L * _NQ
_NW = 32
_KMAX = (_NTASK + _NW - 1) // _NW
_NSLOT = 3


def _sc_add(idx_pad, table, x_t):
    mesh = plsc.VectorSubcoreMesh(core_axis_name="c", subcore_axis_name="s")

    @functools.partial(
        pl.kernel,
        mesh=mesh,
        out_type=jax.ShapeDtypeStruct((_L, _B, _D), jnp.float32),
        scratch_types=[
            pltpu.VMEM((_LPAD,), jnp.int32),
            pltpu.VMEM((_LPAD, _D), jnp.float32),
            pltpu.VMEM((_NSLOT, _CB, _D), jnp.float32),
            pltpu.SemaphoreType.DMA,
            pltpu.SemaphoreType.DMA((_NSLOT,)),
            pltpu.SemaphoreType.DMA((_NSLOT,)),
        ],
    )
    def add_kernel(idx_hbm, table_hbm, x_hbm, out_hbm, idx_v, e_all, buf,
                   gsem, insem, outsem):
        w = lax.axis_index("s") * 2 + lax.axis_index("c")

        pltpu.sync_copy(idx_hbm, idx_v)
        pltpu.async_copy(table_hbm.at[idx_v], e_all, gsem).wait()

        def task(k):
            return w + _NW * k

        def in_copy(k):
            t = task(k)
            s, q = t // _NQ, t % _NQ
            return pltpu.make_async_copy(
                x_hbm.at[s, pl.ds(q * _CB, _CB)], buf.at[k % _NSLOT],
                insem.at[k % _NSLOT])

        def out_copy(k):
            t = task(k)
            s, q = t // _NQ, t % _NQ
            return pltpu.make_async_copy(
                buf.at[k % _NSLOT], out_hbm.at[s, pl.ds(q * _CB, _CB)],
                outsem.at[k % _NSLOT])

        def compute(k):
            t = task(k)
            s = t // _NQ
            bk = buf.at[k % _NSLOT]
            ev = [e_all[s, pl.ds(16 * j, 16)] for j in range(8)]

            @plsc.parallel_loop(0, _CB, unroll=8)
            def _(b):
                for j in range(8):
                    bk[b, pl.ds(16 * j, 16)] = bk[b, pl.ds(16 * j, 16)] + ev[j]

        def step(k):
            if k + 1 < _KMAX:
                guarded(k + 1, lambda kk: in_copy(kk).start())
            in_copy(k).wait()
            compute(k)
            out_copy(k).start()

        def guarded(k, fn):
            if (k + 1) * _NW <= _NTASK:
                fn(k)
            else:
                @pl.when(task(k) < _NTASK)
                def _():
                    fn(k)

        in_copy(0).start()
        for k in range(_KMAX):
            if k >= _NSLOT - 1:
                guarded(k - (_NSLOT - 1), lambda kk: out_copy(kk).wait())
            guarded(k, step)
        for k in range(_KMAX - _NSLOT + 1, _KMAX):
            if k >= 0:
                guarded(k, lambda kk: out_copy(kk).wait())

    return add_kernel(idx_pad, table, x_t)


def kernel(x, variable_idx, variable_embedding):
    idx = variable_idx.astype(jnp.int32)
    idx_pad = jnp.pad(idx, (0, _LPAD - _L))
    x_t = jnp.transpose(x, (1, 0, 2))
    out_t = _sc_add(idx_pad, variable_embedding, x_t)
    return jnp.transpose(out_t, (1, 0, 2))


# speedup vs baseline: 1.0449x; 1.0449x over previous
import functools

import jax
import jax.numpy as jnp
from jax import lax
from jax.experimental import pallas as pl
from jax.experimental.pallas import tpu as pltpu
from jax.experimental.pallas import tpu_sc as plsc

_L = 100
_D = 128
_LPAD = 128
_B = 1024
_CB = 256
_NQ = _B // _CB
_NTASK = _L * _NQ
_NW = 32
_KMAX = (_NTASK + _NW - 1) // _NW
_NSLOT = 3


def _sc_add(idx_pad, table, x_t):
    mesh = plsc.VectorSubcoreMesh(core_axis_name="c", subcore_axis_name="s")

    @functools.partial(
        pl.kernel,
        mesh=mesh,
        out_type=jax.ShapeDtypeStruct((_L, _B, _D), jnp.float32),
        scratch_types=[
            pltpu.VMEM((_LPAD,), jnp.int32),
            pltpu.VMEM((_LPAD, _D), jnp.float32),
            pltpu.VMEM((_NSLOT, _CB, _D), jnp.float32),
            pltpu.SemaphoreType.DMA,
            pltpu.SemaphoreType.DMA((_NSLOT,)),
            pltpu.SemaphoreType.DMA((_NSLOT,)),
        ],
    )
    def add_kernel(idx_hbm, table_hbm, x_hbm, out_hbm, idx_v, e_all, buf,
                   gsem, insem, outsem):
        w = lax.axis_index("s") * 2 + lax.axis_index("c")

        pltpu.sync_copy(idx_hbm, idx_v)
        pltpu.async_copy(table_hbm.at[idx_v], e_all, gsem).wait()

        def task(k):
            return w + _NW * k

        def in_copy(k):
            t = task(k)
            s, q = t // _NQ, t % _NQ
            return pltpu.make_async_copy(
                x_hbm.at[s, pl.ds(q * _CB, _CB)], buf.at[k % _NSLOT],
                insem.at[k % _NSLOT])

        def out_copy(k):
            t = task(k)
            s, q = t // _NQ, t % _NQ
            return pltpu.make_async_copy(
                buf.at[k % _NSLOT], out_hbm.at[s, pl.ds(q * _CB, _CB)],
                outsem.at[k % _NSLOT])

        def compute(k):
            t = task(k)
            s = t // _NQ
            bk = buf.at[k % _NSLOT]
            ev = [e_all[s, pl.ds(16 * j, 16)] for j in range(8)]

            @plsc.parallel_loop(0, _CB, unroll=8)
            def _(b):
                for j in range(8):
                    bk[b, pl.ds(16 * j, 16)] = bk[b, pl.ds(16 * j, 16)] + ev[j]

        def step(k):
            if k + 1 < _KMAX:
                guarded(k + 1, lambda kk: in_copy(kk).start())
            in_copy(k).wait()
            pass
            out_copy(k).start()

        def guarded(k, fn):
            if (k + 1) * _NW <= _NTASK:
                fn(k)
            else:
                @pl.when(task(k) < _NTASK)
                def _():
                    fn(k)

        in_copy(0).start()
        for k in range(_KMAX):
            if k >= _NSLOT - 1:
                guarded(k - (_NSLOT - 1), lambda kk: out_copy(kk).wait())
            guarded(k, step)
        for k in range(_KMAX - _NSLOT + 1, _KMAX):
            if k >= 0:
                guarded(k, lambda kk: out_copy(kk).wait())

    return add_kernel(idx_pad, table, x_t)


def kernel(x, variable_idx, variable_embedding):
    idx = variable_idx.astype(jnp.int32)
    idx_pad = jnp.pad(idx, (0, _LPAD - _L))
    x_t = jnp.transpose(x, (1, 0, 2))
    out_t = _sc_add(idx_pad, variable_embedding, x_t)
    return jnp.transpose(out_t, (1, 0, 2))


# speedup vs baseline: 1.7789x; 1.7025x over previous
import functools

import jax
import jax.numpy as jnp
from jax import lax
from jax.experimental import pallas as pl
from jax.experimental.pallas import tpu as pltpu
from jax.experimental.pallas import tpu_sc as plsc

_L = 100
_D = 128
_LPAD = 128


_GW = 8
_GR = _LPAD // _GW


def _sc_gather(idx_pad, table):
    mesh = plsc.VectorSubcoreMesh(core_axis_name="c", subcore_axis_name="s")

    @functools.partial(
        pl.kernel,
        mesh=mesh,
        out_type=jax.ShapeDtypeStruct((_LPAD, _D), jnp.float32),
        scratch_types=[
            pltpu.VMEM((_GR,), jnp.int32),
            pltpu.VMEM((_GR, _D), jnp.float32),
            pltpu.SemaphoreType.DMA,
        ],
    )
    def gather_kernel(idx_hbm, table_hbm, out_hbm, idx_v, rows_v, sem):
        wid = lax.axis_index("s") * 2 + lax.axis_index("c")

        @pl.when(wid < _GW)
        def _():
            base = wid * _GR
            pltpu.sync_copy(idx_hbm.at[pl.ds(base, _GR)], idx_v)
            pltpu.async_copy(table_hbm.at[idx_v], rows_v, sem).wait()
            pltpu.sync_copy(rows_v, out_hbm.at[pl.ds(base, _GR)])

    return gather_kernel(idx_pad, table)


_B = 1024
_SS = 25


def _add_body(e_ref, x_ref, o_ref):
    o_ref[...] = x_ref[...] + e_ref[...]


def _tc_add_t(x_t, embed3):
    nb = _L // _SS
    return pl.pallas_call(
        _add_body,
        grid=(nb,),
        in_specs=[
            pl.BlockSpec((_SS, 1, _D), lambda i: (i, 0, 0)),
            pl.BlockSpec((_SS, _B, _D), lambda i: (i, 0, 0)),
        ],
        out_specs=pl.BlockSpec((_SS, _B, _D), lambda i: (i, 0, 0)),
        out_shape=jax.ShapeDtypeStruct(x_t.shape, x_t.dtype),
    )(embed3, x_t)


def kernel(x, variable_idx, variable_embedding):
    idx = variable_idx.astype(jnp.int32)
    idx_pad = jnp.pad(idx, (0, _LPAD - _L))
    embed_pad = _sc_gather(idx_pad, variable_embedding)
    embed3 = embed_pad[:_L].reshape(_L, 1, _D)
    x_t = jnp.transpose(x, (1, 0, 2))
    out_t = _tc_add_t(x_t, embed3)
    return jnp.transpose(out_t, (1, 0, 2))


# speedup vs baseline: 1.8439x; 1.0365x over previous
import functools

import jax
import jax.numpy as jnp
from jax import lax
from jax.experimental import pallas as pl
from jax.experimental.pallas import tpu as pltpu
from jax.experimental.pallas import tpu_sc as plsc

_L = 100
_D = 128
_LPAD = 128


_GR = 16


def _sc_gather(idx, table):
    mesh = plsc.VectorSubcoreMesh(core_axis_name="c", subcore_axis_name="s")

    @functools.partial(
        pl.kernel,
        mesh=mesh,
        out_type=jax.ShapeDtypeStruct((_L, 1, _D), jnp.float32),
        scratch_types=[
            pltpu.VMEM((_L,), jnp.int32),
            pltpu.VMEM((_GR, _D), jnp.float32),
            pltpu.SemaphoreType.DMA,
        ],
    )
    def gather_kernel(idx_hbm, table_hbm, out_hbm, idx_v, rows_v, sem):
        wid = lax.axis_index("s") * 2 + lax.axis_index("c")

        for w in range(7):
            base = w * _GR
            cnt = min(_GR, _L - base)

            @pl.when(wid == w)
            def _(base=base, cnt=cnt):
                pltpu.sync_copy(idx_hbm, idx_v)
                pltpu.async_copy(
                    table_hbm.at[idx_v.at[pl.ds(base, cnt)]],
                    rows_v.at[pl.ds(0, cnt)], sem).wait()
                pltpu.sync_copy(rows_v.at[pl.ds(0, cnt)],
                                out_hbm.at[pl.ds(base, cnt), 0, :])

    return gather_kernel(idx, table)


_B = 1024
_SS = 25


def _add_body(e_ref, x_ref, o_ref):
    o_ref[...] = x_ref[...] + e_ref[...]


def _tc_add_t(x_t, embed3):
    nb = _L // _SS
    return pl.pallas_call(
        _add_body,
        grid=(nb,),
        in_specs=[
            pl.BlockSpec((_SS, 1, _D), lambda i: (i, 0, 0)),
            pl.BlockSpec((_SS, _B, _D), lambda i: (i, 0, 0)),
        ],
        out_specs=pl.BlockSpec((_SS, _B, _D), lambda i: (i, 0, 0)),
        out_shape=jax.ShapeDtypeStruct(x_t.shape, x_t.dtype),
    )(embed3, x_t)


def kernel(x, variable_idx, variable_embedding):
    idx = variable_idx.astype(jnp.int32)
    embed3 = _sc_gather(idx, variable_embedding)
    x_t = jnp.transpose(x, (1, 0, 2))
    out_t = _tc_add_t(x_t, embed3)
    return jnp.transpose(out_t, (1, 0, 2))
